# baseline (speedup 1.0000x reference)
import functools

import jax
import jax.numpy as jnp
from jax.experimental import pallas as pl
from jax.experimental.pallas import tpu as pltpu

B = 512
D_MODEL = 1024
D_HIDDEN = 16384
K = 128
DEAD_THRESHOLD = 10000000.0 / 256.0

BH = 1024
NH = D_HIDDEN // BH
BC = 64
NC = B // BC


def _fused_body(x_ref, wenc_ref, wdec_ref, benc_ref, bpre_ref, stats_ref,
                out_ref, ndead_ref,
                keys_ref, xs_ref, mu_ref, std_ref, thr_ref,
                featzero_ref):
    s = pl.program_id(0)

    @pl.when(s == 0)
    def _():
        x = x_ref[...]
        mu = jnp.mean(x, axis=-1, keepdims=True)
        xc = x - mu
        var = jnp.sum(xc * xc, axis=-1, keepdims=True) / (D_MODEL - 1)
        std = jnp.sqrt(var)
        mu_ref[...] = mu
        std_ref[...] = std
        xs_ref[...] = xc / (std + 1e-5) - bpre_ref[...]

    @pl.when(s < NH)
    def _():
        pre = (
            jnp.dot(xs_ref[...], wenc_ref[...], preferred_element_type=jnp.float32)
            + benc_ref[...]
        )
        keys_ref[:, pl.ds(s * BH, BH)] = jax.lax.bitcast_convert_type(
            pre, jnp.int32
        )

    @pl.when((s >= NH) & (s < NH + NC))
    def _():
        c = s - NH
        rows = pl.ds(c * BC, BC)

        thr0 = jnp.zeros((BC, 1), jnp.int32)

        def bit_step(i, thr):
            bit = jnp.int32(1) << (jnp.int32(30) - i)
            cand = thr | bit
            cnt = jnp.sum(
                (keys_ref[rows, :] >= cand).astype(jnp.int32),
                axis=1, keepdims=True,
            )
            return jnp.where(cnt >= K, cand, thr)

        thr_eff = jax.lax.fori_loop(0, 31, bit_step, thr0)
        thr_ref[rows, :] = thr_eff

        chunk_any = jnp.max(
            (keys_ref[rows, :] >= jnp.maximum(thr_eff, 1)).astype(jnp.int32),
            axis=0, keepdims=True,
        )

        @pl.when(c == 0)
        def _():
            featzero_ref[...] = 1 - chunk_any

        @pl.when(c > 0)
        def _():
            featzero_ref[...] = featzero_ref[...] * (1 - chunk_any)

        @pl.when(c == NC - 1)
        def _():
            stats_new = stats_ref[...] * featzero_ref[...] + 1
            dead = (stats_new.astype(jnp.float32) > DEAD_THRESHOLD)
            ndead_ref[0, 0] = jnp.sum(dead.astype(jnp.int32))

    @pl.when(s >= NH + NC)
    def _():
        h = s - (NH + NC)
        key = keys_ref[:, pl.ds(h * BH, BH)]
        lat = jnp.where(
            key >= thr_ref[...],
            jax.lax.bitcast_convert_type(key, jnp.float32),
            0.0,
        )
        part = jnp.dot(lat, wdec_ref[...], preferred_element_type=jnp.float32)

        @pl.when(h == 0)
        def _():
            out_ref[...] = part

        @pl.when(h > 0)
        def _():
            out_ref[...] = out_ref[...] + part

        @pl.when(h == NH - 1)
        def _():
            out_ref[...] = (
                (out_ref[...] + bpre_ref[...]) * std_ref[...] + mu_ref[...]
            )


@functools.partial(jax.jit, static_argnames=("interpret",))
def kernel(x, w_enc, w_dec, b_enc, b_pre, stats_last_nonzero, interpret=False):
    b_enc2 = b_enc.reshape(1, D_HIDDEN)
    b_pre2 = b_pre.reshape(1, D_MODEL)
    stats2 = stats_last_nonzero.reshape(1, D_HIDDEN)

    recons, ndead = pl.pallas_call(
        _fused_body,
        grid=(NH + NC + NH,),
        in_specs=[
            pl.BlockSpec((B, D_MODEL), lambda s: (0, 0)),
            pl.BlockSpec((D_MODEL, BH),
                         lambda s: (0, jnp.where(s < NH, s, NH - 1))),
            pl.BlockSpec((BH, D_MODEL),
                         lambda s: (jnp.where(s >= NH + NC, s - (NH + NC), 0), 0)),
            pl.BlockSpec((1, BH),
                         lambda s: (0, jnp.where(s < NH, s, NH - 1))),
            pl.BlockSpec((1, D_MODEL), lambda s: (0, 0)),
            pl.BlockSpec((1, D_HIDDEN), lambda s: (0, 0)),
        ],
        out_specs=[
            pl.BlockSpec((B, D_MODEL), lambda s: (0, 0)),
            pl.BlockSpec(memory_space=pltpu.SMEM),
        ],
        out_shape=[
            jax.ShapeDtypeStruct((B, D_MODEL), jnp.float32),
            jax.ShapeDtypeStruct((1, 1), jnp.int32),
        ],
        scratch_shapes=[
            pltpu.VMEM((B, D_HIDDEN), jnp.int32),
            pltpu.VMEM((B, D_MODEL), jnp.float32),
            pltpu.VMEM((B, 1), jnp.float32),
            pltpu.VMEM((B, 1), jnp.float32),
            pltpu.VMEM((B, 1), jnp.int32),
            pltpu.VMEM((1, D_HIDDEN), jnp.int32),
        ],
        compiler_params=pltpu.CompilerParams(
            dimension_semantics=("arbitrary",),
            vmem_limit_bytes=63 * 1024 * 1024,
        ),
        interpret=interpret,
    )(x, w_enc, w_dec, b_enc2, b_pre2, stats2)

    return (recons, ndead[0, 0])


# speedup vs baseline: 1.1004x; 1.1004x over previous
import functools

import jax
import jax.numpy as jnp
from jax.experimental import pallas as pl
from jax.experimental.pallas import tpu as pltpu

B = 512
D_MODEL = 1024
D_HIDDEN = 16384
K = 128
DEAD_THRESHOLD = 10000000.0 / 256.0

BH = 1024
NH = D_HIDDEN // BH
BC = 128
NC = B // BC


def _fused_body(x_ref, wenc_ref, wdec_ref, benc_ref, bpre_ref, stats_ref,
                out_ref, ndead_ref,
                keys_ref, xs_ref, mu_ref, std_ref, thr_ref,
                featzero_ref):
    s = pl.program_id(0)

    @pl.when(s == 0)
    def _():
        x = x_ref[...]
        mu = jnp.mean(x, axis=-1, keepdims=True)
        xc = x - mu
        var = jnp.sum(xc * xc, axis=-1, keepdims=True) / (D_MODEL - 1)
        std = jnp.sqrt(var)
        mu_ref[...] = mu
        std_ref[...] = std
        xs_ref[...] = xc / (std + 1e-5) - bpre_ref[...]

    @pl.when(s < NH)
    def _():
        pre = (
            jnp.dot(xs_ref[...], wenc_ref[...], preferred_element_type=jnp.float32)
            + benc_ref[...]
        )
        keys_ref[:, pl.ds(s * BH, BH)] = jax.lax.bitcast_convert_type(
            pre, jnp.int32
        )

    @pl.when((s >= NH) & (s < NH + NC))
    def _():
        c = s - NH
        rows = pl.ds(c * BC, BC)

        thr0 = jnp.zeros((BC, 1), jnp.int32)

        def bit_step(i, thr):
            bit = jnp.int32(1) << (jnp.int32(30) - i)
            cand = thr | bit
            cnt = jnp.sum(
                (keys_ref[rows, :] >= cand).astype(jnp.int32),
                axis=1, keepdims=True,
            )
            return jnp.where(cnt >= K, cand, thr)

        thr_eff = jax.lax.fori_loop(0, 31, bit_step, thr0)
        thr_ref[rows, :] = thr_eff

        chunk_any = jnp.max(
            (keys_ref[rows, :] >= jnp.maximum(thr_eff, 1)).astype(jnp.int32),
            axis=0, keepdims=True,
        )

        @pl.when(c == 0)
        def _():
            featzero_ref[...] = 1 - chunk_any

        @pl.when(c > 0)
        def _():
            featzero_ref[...] = featzero_ref[...] * (1 - chunk_any)

        @pl.when(c == NC - 1)
        def _():
            stats_new = stats_ref[...] * featzero_ref[...] + 1
            dead = (stats_new.astype(jnp.float32) > DEAD_THRESHOLD)
            ndead_ref[0, 0] = jnp.sum(dead.astype(jnp.int32))

    @pl.when(s >= NH + NC)
    def _():
        h = s - (NH + NC)
        key = keys_ref[:, pl.ds(h * BH, BH)]
        lat = jnp.where(
            key >= thr_ref[...],
            jax.lax.bitcast_convert_type(key, jnp.float32),
            0.0,
        )
        part = jnp.dot(lat, wdec_ref[...], preferred_element_type=jnp.float32)

        @pl.when(h == 0)
        def _():
            out_ref[...] = part

        @pl.when(h > 0)
        def _():
            out_ref[...] = out_ref[...] + part

        @pl.when(h == NH - 1)
        def _():
            out_ref[...] = (
                (out_ref[...] + bpre_ref[...]) * std_ref[...] + mu_ref[...]
            )


@functools.partial(jax.jit, static_argnames=("interpret",))
def kernel(x, w_enc, w_dec, b_enc, b_pre, stats_last_nonzero, interpret=False):
    b_enc2 = b_enc.reshape(1, D_HIDDEN)
    b_pre2 = b_pre.reshape(1, D_MODEL)
    stats2 = stats_last_nonzero.reshape(1, D_HIDDEN)

    recons, ndead = pl.pallas_call(
        _fused_body,
        grid=(NH + NC + NH,),
        in_specs=[
            pl.BlockSpec((B, D_MODEL), lambda s: (0, 0)),
            pl.BlockSpec((D_MODEL, BH),
                         lambda s: (0, jnp.where(s < NH, s, NH - 1))),
            pl.BlockSpec((BH, D_MODEL),
                         lambda s: (jnp.where(s >= NH + NC, s - (NH + NC), 0), 0)),
            pl.BlockSpec((1, BH),
                         lambda s: (0, jnp.where(s < NH, s, NH - 1))),
            pl.BlockSpec((1, D_MODEL), lambda s: (0, 0)),
            pl.BlockSpec((1, D_HIDDEN), lambda s: (0, 0)),
        ],
        out_specs=[
            pl.BlockSpec((B, D_MODEL), lambda s: (0, 0)),
            pl.BlockSpec(memory_space=pltpu.SMEM),
        ],
        out_shape=[
            jax.ShapeDtypeStruct((B, D_MODEL), jnp.float32),
            jax.ShapeDtypeStruct((1, 1), jnp.int32),
        ],
        scratch_shapes=[
            pltpu.VMEM((B, D_HIDDEN), jnp.int32),
            pltpu.VMEM((B, D_MODEL), jnp.float32),
            pltpu.VMEM((B, 1), jnp.float32),
            pltpu.VMEM((B, 1), jnp.float32),
            pltpu.VMEM((B, 1), jnp.int32),
            pltpu.VMEM((1, D_HIDDEN), jnp.int32),
        ],
        compiler_params=pltpu.CompilerParams(
            dimension_semantics=("arbitrary",),
            vmem_limit_bytes=63 * 1024 * 1024,
        ),
        interpret=interpret,
    )(x, w_enc, w_dec, b_enc2, b_pre2, stats2)

    return (recons, ndead[0, 0])
